# baseline (speedup 1.0000x reference)
import jax
import jax.numpy as jnp
from jax.experimental import pallas as pl
from jax.experimental.pallas import tpu as pltpu

_GROUPS = 8
_BM = 1000


def _mlp_body(x_ref, w1_ref, b1_ref, w2_ref, b2_ref, o_ref):
    o = jnp.broadcast_to(b2_ref[...], o_ref.shape)
    w1 = w1_ref[...]
    b1 = b1_ref[...]
    for t in range(_GROUPS):
        xt = x_ref[:, 128 * t:128 * (t + 1)]
        z = jnp.dot(xt, w1, preferred_element_type=jnp.float32) + b1
        h = jnp.maximum(z, 0.01 * z)
        o = o + jnp.dot(h, w2_ref[128 * t:128 * (t + 1), :],
                        preferred_element_type=jnp.float32)
    o_ref[...] = o


@jax.jit
def kernel(X, W1, b1, W2, b2):
    n, d = X.shape
    width = 128 * _GROUPS
    rows = (n * d) // width
    xr = X.reshape(rows, width)

    eye8 = jnp.eye(8, dtype=X.dtype)
    w1bd = jnp.kron(eye8, W1)
    b1bd = jnp.tile(b1, 8).reshape(1, 128)
    eye64 = jnp.eye(64, dtype=X.dtype)
    w2w = jnp.concatenate(
        [jnp.kron(eye64, W2[:, 0:1]), jnp.kron(eye64, W2[:, 1:2])], axis=1
    )
    b2w = jnp.concatenate(
        [jnp.full((64,), b2[0], X.dtype), jnp.full((64,), b2[1], X.dtype)]
    ).reshape(1, 128)

    grid = (pl.cdiv(rows, _BM),)
    out = pl.pallas_call(
        _mlp_body,
        grid=grid,
        in_specs=[
            pl.BlockSpec((_BM, width), lambda i: (i, 0)),
            pl.BlockSpec((128, 128), lambda i: (0, 0)),
            pl.BlockSpec((1, 128), lambda i: (0, 0)),
            pl.BlockSpec((width, 128), lambda i: (0, 0)),
            pl.BlockSpec((1, 128), lambda i: (0, 0)),
        ],
        out_specs=pl.BlockSpec((_BM, 128), lambda i: (i, 0)),
        out_shape=jax.ShapeDtypeStruct((rows, 128), X.dtype),
        compiler_params=pltpu.CompilerParams(
            dimension_semantics=("parallel",),
        ),
    )(xr, w1bd, b1bd, w2w, b2w)
    return out[:, 0:64].reshape(n), out[:, 64:128].reshape(n)


# speedup vs baseline: 6.2842x; 6.2842x over previous
import jax
import jax.numpy as jnp
from jax.experimental import pallas as pl
from jax.experimental.pallas import tpu as pltpu

_BN = 8192


def _mlp_body(x_ref, w1_ref, b1_ref, w2_ref, b2_ref, la_ref, lb_ref):
    x = x_ref[...]
    z = jnp.dot(w1_ref[...], x, preferred_element_type=jnp.float32)
    z = z + b1_ref[...]
    h = jnp.maximum(z, 0.01 * z)
    o = jnp.dot(w2_ref[...], h, preferred_element_type=jnp.float32)
    o = o + b2_ref[...]
    la_ref[...] = o[0, :]
    lb_ref[...] = o[1, :]


@jax.jit
def kernel(X, W1, b1, W2, b2):
    n, d = X.shape
    xt = X.T

    w1t = W1.T
    b1c = b1.reshape(d, 1)
    w2t = jnp.concatenate(
        [W2.T, jnp.zeros((8 - W2.shape[1], d), W2.dtype)], axis=0
    )
    b2c = jnp.concatenate(
        [b2, jnp.zeros((8 - b2.shape[0],), b2.dtype)]
    ).reshape(8, 1)

    grid = (pl.cdiv(n, _BN),)
    la, lb = pl.pallas_call(
        _mlp_body,
        grid=grid,
        in_specs=[
            pl.BlockSpec((d, _BN), lambda j: (0, j)),
            pl.BlockSpec((d, d), lambda j: (0, 0)),
            pl.BlockSpec((d, 1), lambda j: (0, 0)),
            pl.BlockSpec((8, d), lambda j: (0, 0)),
            pl.BlockSpec((8, 1), lambda j: (0, 0)),
        ],
        out_specs=[
            pl.BlockSpec((_BN,), lambda j: (j,)),
            pl.BlockSpec((_BN,), lambda j: (j,)),
        ],
        out_shape=[
            jax.ShapeDtypeStruct((n,), X.dtype),
            jax.ShapeDtypeStruct((n,), X.dtype),
        ],
        compiler_params=pltpu.CompilerParams(
            dimension_semantics=("arbitrary",),
        ),
    )(xt, w1t, b1c, w2t, b2c)
    return la, lb


# speedup vs baseline: 13.9380x; 2.2179x over previous
import jax
import jax.numpy as jnp
from jax.experimental import pallas as pl
from jax.experimental.pallas import tpu as pltpu

_BN = 32768


def _mlp_body(x_ref, w1_ref, b1_ref, w2_ref, b2_ref, la_ref, lb_ref):
    x = x_ref[...]
    z = jnp.dot(w1_ref[...], x, preferred_element_type=jnp.float32)
    z = z + b1_ref[...]
    h = jnp.maximum(z, 0.01 * z)
    o = jnp.dot(w2_ref[...], h, preferred_element_type=jnp.float32)
    o = o + b2_ref[...]
    la_ref[...] = o[0, :]
    lb_ref[...] = o[1, :]


@jax.jit
def kernel(X, W1, b1, W2, b2):
    n, d = X.shape
    xt = X.T

    w1t = W1.T
    b1c = b1.reshape(d, 1)
    w2t = jnp.concatenate(
        [W2.T, jnp.zeros((8 - W2.shape[1], d), W2.dtype)], axis=0
    )
    b2c = jnp.concatenate(
        [b2, jnp.zeros((8 - b2.shape[0],), b2.dtype)]
    ).reshape(8, 1)

    grid = (pl.cdiv(n, _BN),)
    la, lb = pl.pallas_call(
        _mlp_body,
        grid=grid,
        in_specs=[
            pl.BlockSpec((d, _BN), lambda j: (0, j)),
            pl.BlockSpec((d, d), lambda j: (0, 0)),
            pl.BlockSpec((d, 1), lambda j: (0, 0)),
            pl.BlockSpec((8, d), lambda j: (0, 0)),
            pl.BlockSpec((8, 1), lambda j: (0, 0)),
        ],
        out_specs=[
            pl.BlockSpec((_BN,), lambda j: (j,)),
            pl.BlockSpec((_BN,), lambda j: (j,)),
        ],
        out_shape=[
            jax.ShapeDtypeStruct((n,), X.dtype),
            jax.ShapeDtypeStruct((n,), X.dtype),
        ],
        compiler_params=pltpu.CompilerParams(
            dimension_semantics=("arbitrary",),
        ),
    )(xt, w1t, b1c, w2t, b2c)
    return la, lb


# speedup vs baseline: 17.0899x; 1.2261x over previous
import jax
import jax.numpy as jnp
from jax.experimental import pallas as pl
from jax.experimental.pallas import tpu as pltpu

_BN = 65536


def _mlp_body(x_ref, w1_ref, b1_ref, w2_ref, b2_ref, la_ref, lb_ref):
    x = x_ref[...]
    z = jnp.dot(w1_ref[...], x, preferred_element_type=jnp.float32)
    z = z + b1_ref[...]
    h = jnp.maximum(z, 0.01 * z)
    o = jnp.dot(w2_ref[...], h, preferred_element_type=jnp.float32)
    o = o + b2_ref[...]
    la_ref[...] = o[0, :]
    lb_ref[...] = o[1, :]


@jax.jit
def kernel(X, W1, b1, W2, b2):
    n, d = X.shape
    xt = X.T

    w1t = W1.T
    b1c = b1.reshape(d, 1)
    w2t = jnp.concatenate(
        [W2.T, jnp.zeros((8 - W2.shape[1], d), W2.dtype)], axis=0
    )
    b2c = jnp.concatenate(
        [b2, jnp.zeros((8 - b2.shape[0],), b2.dtype)]
    ).reshape(8, 1)

    grid = (pl.cdiv(n, _BN),)
    la, lb = pl.pallas_call(
        _mlp_body,
        grid=grid,
        in_specs=[
            pl.BlockSpec((d, _BN), lambda j: (0, j)),
            pl.BlockSpec((d, d), lambda j: (0, 0)),
            pl.BlockSpec((d, 1), lambda j: (0, 0)),
            pl.BlockSpec((8, d), lambda j: (0, 0)),
            pl.BlockSpec((8, 1), lambda j: (0, 0)),
        ],
        out_specs=[
            pl.BlockSpec((_BN,), lambda j: (j,)),
            pl.BlockSpec((_BN,), lambda j: (j,)),
        ],
        out_shape=[
            jax.ShapeDtypeStruct((n,), X.dtype),
            jax.ShapeDtypeStruct((n,), X.dtype),
        ],
        compiler_params=pltpu.CompilerParams(
            dimension_semantics=("arbitrary",),
        ),
    )(xt, w1t, b1c, w2t, b2c)
    return la, lb


# speedup vs baseline: 18.7849x; 1.0992x over previous
import jax
import jax.numpy as jnp
from jax.experimental import pallas as pl
from jax.experimental.pallas import tpu as pltpu

_BN = 131072


def _mlp_body(x_ref, w1_ref, b1_ref, w2_ref, b2_ref, la_ref, lb_ref):
    x = x_ref[...]
    z = jnp.dot(w1_ref[...], x, preferred_element_type=jnp.float32)
    z = z + b1_ref[...]
    h = jnp.maximum(z, 0.01 * z)
    o = jnp.dot(w2_ref[...], h, preferred_element_type=jnp.float32)
    o = o + b2_ref[...]
    la_ref[...] = o[0, :]
    lb_ref[...] = o[1, :]


@jax.jit
def kernel(X, W1, b1, W2, b2):
    n, d = X.shape
    xt = X.T

    w1t = W1.T
    b1c = b1.reshape(d, 1)
    w2t = jnp.concatenate(
        [W2.T, jnp.zeros((8 - W2.shape[1], d), W2.dtype)], axis=0
    )
    b2c = jnp.concatenate(
        [b2, jnp.zeros((8 - b2.shape[0],), b2.dtype)]
    ).reshape(8, 1)

    grid = (pl.cdiv(n, _BN),)
    la, lb = pl.pallas_call(
        _mlp_body,
        grid=grid,
        in_specs=[
            pl.BlockSpec((d, _BN), lambda j: (0, j)),
            pl.BlockSpec((d, d), lambda j: (0, 0)),
            pl.BlockSpec((d, 1), lambda j: (0, 0)),
            pl.BlockSpec((8, d), lambda j: (0, 0)),
            pl.BlockSpec((8, 1), lambda j: (0, 0)),
        ],
        out_specs=[
            pl.BlockSpec((_BN,), lambda j: (j,)),
            pl.BlockSpec((_BN,), lambda j: (j,)),
        ],
        out_shape=[
            jax.ShapeDtypeStruct((n,), X.dtype),
            jax.ShapeDtypeStruct((n,), X.dtype),
        ],
        compiler_params=pltpu.CompilerParams(
            dimension_semantics=("arbitrary",),
        ),
    )(xt, w1t, b1c, w2t, b2c)
    return la, lb


# speedup vs baseline: 19.0728x; 1.0153x over previous
import jax
import jax.numpy as jnp
from jax.experimental import pallas as pl
from jax.experimental.pallas import tpu as pltpu

_BN = 262144


def _mlp_body(x_ref, w1_ref, b1_ref, w2_ref, b2_ref, la_ref, lb_ref):
    x = x_ref[...]
    z = jnp.dot(w1_ref[...], x, preferred_element_type=jnp.float32)
    z = z + b1_ref[...]
    h = jnp.maximum(z, 0.01 * z)
    o = jnp.dot(w2_ref[...], h, preferred_element_type=jnp.float32)
    o = o + b2_ref[...]
    la_ref[...] = o[0, :]
    lb_ref[...] = o[1, :]


@jax.jit
def kernel(X, W1, b1, W2, b2):
    n, d = X.shape
    xt = X.T

    w1t = W1.T
    b1c = b1.reshape(d, 1)
    w2t = jnp.concatenate(
        [W2.T, jnp.zeros((8 - W2.shape[1], d), W2.dtype)], axis=0
    )
    b2c = jnp.concatenate(
        [b2, jnp.zeros((8 - b2.shape[0],), b2.dtype)]
    ).reshape(8, 1)

    grid = (pl.cdiv(n, _BN),)
    la, lb = pl.pallas_call(
        _mlp_body,
        grid=grid,
        in_specs=[
            pl.BlockSpec((d, _BN), lambda j: (0, j)),
            pl.BlockSpec((d, d), lambda j: (0, 0)),
            pl.BlockSpec((d, 1), lambda j: (0, 0)),
            pl.BlockSpec((8, d), lambda j: (0, 0)),
            pl.BlockSpec((8, 1), lambda j: (0, 0)),
        ],
        out_specs=[
            pl.BlockSpec((_BN,), lambda j: (j,)),
            pl.BlockSpec((_BN,), lambda j: (j,)),
        ],
        out_shape=[
            jax.ShapeDtypeStruct((n,), X.dtype),
            jax.ShapeDtypeStruct((n,), X.dtype),
        ],
        compiler_params=pltpu.CompilerParams(
            dimension_semantics=("arbitrary",),
        ),
    )(xt, w1t, b1c, w2t, b2c)
    return la, lb
